# baseline (speedup 1.0000x reference)
import functools

import jax
import jax.numpy as jnp
from jax import lax
from jax.experimental import pallas as pl
from jax.experimental.pallas import tpu as pltpu
from jax.experimental.pallas import tpu_sc as plsc

D = 32
L = 16
IC = 128
W = 128


@functools.lru_cache(maxsize=None)
def _build(B, NU4, NB):
    info = plsc.get_sparse_core_info()
    NC, NS = info.num_cores, info.num_subcores
    NW = NC * NS
    BW = B // NW
    NCH = BW // IC
    mesh = plsc.VectorSubcoreMesh(core_axis_name="c", subcore_axis_name="s")

    idx_scratch = []
    for _ in range(NCH):
        idx_scratch += [
            pltpu.VMEM((IC,), jnp.int32),
            pltpu.VMEM((IC,), jnp.int32),
            pltpu.VMEM((IC,), jnp.int32),
            pltpu.VMEM((IC,), jnp.int32),
            pltpu.VMEM((IC,), jnp.int32),
            pltpu.VMEM((IC,), jnp.int32),
        ]

    @functools.partial(
        pl.kernel,
        mesh=mesh,
        compiler_params=pltpu.CompilerParams(needs_layout_passes=False),
        out_type=jax.ShapeDtypeStruct((B,), jnp.float32),
        scratch_types=idx_scratch + [
            pltpu.VMEM((IC, W), jnp.float32),
            pltpu.VMEM((IC, W), jnp.float32),
            pltpu.VMEM((IC, W), jnp.float32),
            pltpu.VMEM((IC, W), jnp.float32),
            pltpu.VMEM((L,), jnp.float32),
            pltpu.VMEM((BW,), jnp.float32),
            pltpu.SemaphoreType.DMA,
        ],
    )
    def mf_kernel(u_hbm, i_hbm, U_hbm, V_hbm, bu_hbm, bi_hbm, gb_hbm,
                  out_hbm, *refs):
        idx = refs[:6 * NCH]
        urows, vrows, burows, birows, gbv, outb, sem = refs[6 * NCH:]
        wid = lax.axis_index("s") * NC + lax.axis_index("c")
        base = wid * BW
        pltpu.sync_copy(gb_hbm, gbv)
        iota = lax.iota(jnp.int32, L)

        for j in range(NCH):
            uj, ij, uej, iej, ubj, ibj = idx[6 * j:6 * j + 6]
            pltpu.sync_copy(u_hbm.at[pl.ds(base + j * IC, IC)], uj)
            pltpu.sync_copy(i_hbm.at[pl.ds(base + j * IC, IC)], ij)
            for t in range(IC // L):
                sl = t * L + iota
                uv = plsc.load_gather(uj, [sl])
                iv = plsc.load_gather(ij, [sl])
                plsc.store_scatter(uej, [sl], lax.shift_right_logical(uv, 2))
                plsc.store_scatter(iej, [sl], lax.shift_right_logical(iv, 2))
                plsc.store_scatter(ubj, [sl], lax.shift_right_logical(uv, 7))
                plsc.store_scatter(ibj, [sl], lax.shift_right_logical(iv, 7))

        gb = gbv[...]
        for j in range(NCH):
            uj, ij, uej, iej, ubj, ibj = idx[6 * j:6 * j + 6]
            cs = [pltpu.async_copy(U_hbm.at[uej], urows, sem),
                  pltpu.async_copy(V_hbm.at[iej], vrows, sem),
                  pltpu.async_copy(bu_hbm.at[ubj], burows, sem),
                  pltpu.async_copy(bi_hbm.at[ibj], birows, sem)]
            for c in cs:
                c.wait()

            def group(g, carry, _uj=uj, _ij=ij, _j=j):
                rows = g * L + iota
                uraw = plsc.load_gather(_uj, [rows])
                iraw = plsc.load_gather(_ij, [rows])
                ucol = (uraw & 3) * D
                icol = (iraw & 3) * D
                acc = gb
                for d in range(D):
                    acc = acc + (plsc.load_gather(urows, [rows, ucol + d])
                                 * plsc.load_gather(vrows, [rows, icol + d]))
                acc = (acc + plsc.load_gather(burows, [rows, uraw & 127])
                       + plsc.load_gather(birows, [rows, iraw & 127]))
                res = 1.0 + 4.0 / (1.0 + jnp.exp(-acc))
                plsc.store_scatter(outb, [_j * IC + rows], res)
                return carry

            lax.fori_loop(0, IC // L, group, 0)

        pltpu.sync_copy(outb, out_hbm.at[pl.ds(base, BW)])

    return mf_kernel


def kernel(u, i, U, V, bu, bi, global_b):
    B = u.shape[0]
    NU, _ = U.shape
    NB = -(-NU // W)
    pad = NB * W - NU
    f = _build(B, NU // 4, NB)
    U128 = U.reshape(NU // 4, W)
    V128 = V.reshape(NU // 4, W)
    bup = jnp.pad(bu.reshape(-1), (0, pad)).reshape(NB, W)
    bip = jnp.pad(bi.reshape(-1), (0, pad)).reshape(NB, W)
    gb = jnp.broadcast_to(global_b.astype(jnp.float32), (L,))
    return f(u.astype(jnp.int32), i.astype(jnp.int32), U128, V128,
             bup, bip, gb)


# speedup vs baseline: 1.0943x; 1.0943x over previous
import functools

import jax
import jax.numpy as jnp
from jax import lax
from jax.experimental import pallas as pl
from jax.experimental.pallas import tpu as pltpu
from jax.experimental.pallas import tpu_sc as plsc

D = 32
L = 16
IC = 128
W = 128


@functools.lru_cache(maxsize=None)
def _build(B, NU4, NB):
    info = plsc.get_sparse_core_info()
    NC, NS = info.num_cores, info.num_subcores
    NW = NC * NS
    BW = B // NW
    NCH = BW // IC
    mesh = plsc.VectorSubcoreMesh(core_axis_name="c", subcore_axis_name="s")

    idx_scratch = []
    for _ in range(NCH):
        idx_scratch += [
            pltpu.VMEM((IC,), jnp.int32),
            pltpu.VMEM((IC,), jnp.int32),
            pltpu.VMEM((IC,), jnp.int32),
            pltpu.VMEM((IC,), jnp.int32),
            pltpu.VMEM((IC,), jnp.int32),
            pltpu.VMEM((IC,), jnp.int32),
        ]

    @functools.partial(
        pl.kernel,
        mesh=mesh,
        compiler_params=pltpu.CompilerParams(needs_layout_passes=False),
        out_type=jax.ShapeDtypeStruct((B,), jnp.float32),
        scratch_types=idx_scratch + [
            pltpu.VMEM((IC, W), jnp.float32),
            pltpu.VMEM((IC, W), jnp.float32),
            pltpu.VMEM((IC, W), jnp.float32),
            pltpu.VMEM((IC, W), jnp.float32),
            pltpu.VMEM((L,), jnp.float32),
            pltpu.VMEM((BW,), jnp.float32),
            pltpu.SemaphoreType.DMA,
        ],
    )
    def mf_kernel(u_hbm, i_hbm, U_hbm, V_hbm, bu_hbm, bi_hbm, gb_hbm,
                  out_hbm, *refs):
        idx = refs[:6 * NCH]
        urows, vrows, burows, birows, gbv, outb, sem = refs[6 * NCH:]
        wid = lax.axis_index("s") * NC + lax.axis_index("c")
        base = wid * BW
        pltpu.sync_copy(gb_hbm, gbv)
        iota = lax.iota(jnp.int32, L)

        for j in range(NCH):
            uj, ij, uej, iej, ubj, ibj = idx[6 * j:6 * j + 6]
            pltpu.sync_copy(u_hbm.at[pl.ds(base + j * IC, IC)], uj)
            pltpu.sync_copy(i_hbm.at[pl.ds(base + j * IC, IC)], ij)
            for t in range(IC // L):
                sl = t * L + iota
                uv = plsc.load_gather(uj, [sl])
                iv = plsc.load_gather(ij, [sl])
                plsc.store_scatter(uej, [sl], lax.shift_right_logical(uv, 2))
                plsc.store_scatter(iej, [sl], lax.shift_right_logical(iv, 2))
                plsc.store_scatter(ubj, [sl], lax.shift_right_logical(uv, 7))
                plsc.store_scatter(ibj, [sl], lax.shift_right_logical(iv, 7))

        gb = gbv[...]
        for j in range(NCH):
            uj, ij, uej, iej, ubj, ibj = idx[6 * j:6 * j + 6]
            cs = [pltpu.async_copy(U_hbm.at[uej], urows, sem),
                  pltpu.async_copy(V_hbm.at[iej], vrows, sem),
                  pltpu.async_copy(bu_hbm.at[ubj], burows, sem),
                  pltpu.async_copy(bi_hbm.at[ibj], birows, sem)]
            for c in cs:
                c.wait()

            def group(g, carry, _uj=uj, _ij=ij, _j=j):
                rows = g * L + iota
                uraw = plsc.load_gather(_uj, [rows])
                iraw = plsc.load_gather(_ij, [rows])
                ucol = (uraw & 3) * D
                icol = (iraw & 3) * D
                acc = gb
                for d in range(D):
                    acc = acc + (plsc.load_gather(urows, [rows, ucol + d])
                                 * plsc.load_gather(vrows, [rows, icol + d]))
                acc = (acc + plsc.load_gather(burows, [rows, uraw & 127])
                       + plsc.load_gather(birows, [rows, iraw & 127]))
                res = 1.0 + 4.0 / (1.0 + jnp.exp(-acc))
                plsc.store_scatter(outb, [_j * IC + rows], res)
                return carry

            lax.fori_loop(0, IC // L, group, 0)

        pltpu.sync_copy(outb, out_hbm.at[pl.ds(base, BW)])

    return mf_kernel


def kernel(u, i, U, V, bu, bi, global_b):
    B = u.shape[0]
    NU, _ = U.shape
    NB = -(-NU // W)
    pad = NB * W - NU
    f = _build(B, NU // 4, NB)
    def rows128(T):
        return (T.T.reshape(D, NU // 4, 4).transpose(1, 2, 0)
                .reshape(NU // 4, W))

    U128 = rows128(U)
    V128 = rows128(V)
    bup = jnp.pad(bu, ((0, pad), (0, 0))).reshape(NB, W)
    bip = jnp.pad(bi, ((0, pad), (0, 0))).reshape(NB, W)
    gb = jnp.broadcast_to(global_b.astype(jnp.float32), (L,))
    return f(u.astype(jnp.int32), i.astype(jnp.int32), U128, V128,
             bup, bip, gb)
